# baseline (speedup 1.0000x reference)
import functools

import jax
import jax.numpy as jnp
from jax import lax
from jax.experimental import pallas as pl
from jax.experimental.pallas import tpu as pltpu
from jax.experimental.pallas import tpu_sc as plsc

B, V, C, S = 4096, 100, 64, 1000
BV = B * V
NSPLIT = 2
BVH = BV // NSPLIT
NC, NS, L = 2, 16, 16
NW = NC * NS
PER = BVH // NW
CHUNK = 128
NCHUNK = PER // CHUNK
NBUF = 10


@functools.partial(
    pl.kernel,
    out_type=jax.ShapeDtypeStruct((BVH, C), jnp.float32),
    mesh=plsc.VectorSubcoreMesh(core_axis_name="c", subcore_axis_name="s"),
    scratch_types=[
        pltpu.VMEM((NCHUNK, CHUNK), jnp.int32),
        pltpu.VMEM((NBUF, CHUNK, C), jnp.float32),
        pltpu.SemaphoreType.DMA((NBUF,)),
        pltpu.SemaphoreType.DMA,
    ],
    compiler_params=pltpu.CompilerParams(use_tc_tiling_on_sc=False),
)
def _sc_gather(x_hbm, tab_hbm, out_hbm, idx_v, rows_v, gsem, ssem):
    wid = lax.axis_index("s") * NC + lax.axis_index("c")
    base = wid * PER
    pltpu.sync_copy(x_hbm.at[wid], idx_v)
    lane = lax.iota(jnp.int32, 16)

    def to_indices(r):
        rowbase = base + r * CHUNK
        for c in range(CHUNK // L):
            f = rowbase + c * L + lane
            xv = idx_v[r, pl.ds(c * L, L)]
            idx_v[r, pl.ds(c * L, L)] = (f % V) * S + xv

    def fire_gather(j, b):
        pltpu.async_copy(tab_hbm.at[idx_v.at[j]], rows_v.at[b], gsem.at[b])

    def wait_gather(j, b):
        pltpu.make_async_copy(
            tab_hbm.at[idx_v.at[j]], rows_v.at[b], gsem.at[b]).wait()

    def store(j, b):
        pltpu.async_copy(
            rows_v.at[b], out_hbm.at[pl.ds(base + j * CHUNK, CHUNK)], ssem
        ).wait()

    for b in range(NBUF):
        to_indices(b)
        fire_gather(b, b)

    def transform_rest(r, carry):
        to_indices(r)
        return carry

    lax.fori_loop(NBUF, NCHUNK, transform_rest, 0)

    def round_fn(gi, carry):
        g = gi * NBUF
        for b in range(NBUF):
            j = g + b
            wait_gather(j, b)
            store(j, b)
            fire_gather(j + NBUF, b)
        return carry

    lax.fori_loop(0, (NCHUNK - NBUF) // NBUF, round_fn, 0)

    for b in range(NBUF):
        j = NCHUNK - NBUF + b
        wait_gather(j, b)
        store(j, b)


def kernel(x, weight):
    tab = jnp.transpose(weight.reshape(V, C, S), (0, 2, 1)).reshape(V * S, C)
    xs = x.reshape(NSPLIT, NW, NCHUNK, CHUNK)
    halves = [_sc_gather(xs[i], tab) for i in range(NSPLIT)]
    return jnp.concatenate(halves, axis=0).reshape(B, V, 1, C)


# speedup vs baseline: 1.7681x; 1.7681x over previous
import functools

import jax
import jax.numpy as jnp
from jax import lax
from jax.experimental import pallas as pl
from jax.experimental.pallas import tpu as pltpu
from jax.experimental.pallas import tpu_sc as plsc

B, V, C, S = 4096, 100, 64, 1000
BV = B * V
VS = V * S
NC, NS, L = 2, 16, 16
NW = NC * NS
PER = BV // NW
CHUNK = 128
NCHUNK = PER // CHUNK
NBUF = 10


@functools.partial(
    pl.kernel,
    out_type=jax.ShapeDtypeStruct((BV, C), jnp.float32),
    mesh=plsc.VectorSubcoreMesh(core_axis_name="c", subcore_axis_name="s"),
    scratch_types=[
        pltpu.VMEM((PER,), jnp.int32),
        pltpu.VMEM((NBUF, CHUNK, C), jnp.float32),
        pltpu.SemaphoreType.DMA((NBUF,)),
        pltpu.SemaphoreType.DMA,
    ],
    compiler_params=pltpu.CompilerParams(use_tc_tiling_on_sc=False),
)
def _sc_gather(x_hbm, tab_hbm, out_hbm, idx_v, rows_v, gsem, ssem):
    wid = lax.axis_index("s") * NC + lax.axis_index("c")
    base = wid * PER
    pltpu.sync_copy(x_hbm.at[pl.ds(base, PER)], idx_v)
    lane = lax.iota(jnp.int32, 16)

    def to_indices(r):
        rowbase = base + r * CHUNK
        for c in range(CHUNK // L):
            f = rowbase + c * L + lane
            xv = idx_v[pl.ds(r * CHUNK + c * L, L)]
            idx_v[pl.ds(r * CHUNK + c * L, L)] = (f % V) * S + xv

    def fire_gather(j, b):
        pltpu.async_copy(
            tab_hbm.at[idx_v.at[pl.ds(j * CHUNK, CHUNK)]], rows_v.at[b],
            gsem.at[b])

    def wait_gather(j, b):
        pltpu.make_async_copy(
            tab_hbm.at[idx_v.at[pl.ds(j * CHUNK, CHUNK)]], rows_v.at[b],
            gsem.at[b]).wait()

    def store(j, b):
        pltpu.async_copy(
            rows_v.at[b], out_hbm.at[pl.ds(base + j * CHUNK, CHUNK)], ssem
        ).wait()

    for b in range(NBUF):
        to_indices(b)
        fire_gather(b, b)

    def transform_rest(r, carry):
        to_indices(r)
        return carry

    lax.fori_loop(NBUF, NCHUNK, transform_rest, 0)

    def round_fn(gi, carry):
        g = gi * NBUF
        for b in range(NBUF):
            j = g + b
            wait_gather(j, b)
            store(j, b)
            fire_gather(j + NBUF, b)
        return carry

    lax.fori_loop(0, (NCHUNK - NBUF) // NBUF, round_fn, 0)

    for b in range(NBUF):
        j = NCHUNK - NBUF + b
        wait_gather(j, b)
        store(j, b)


def kernel(x, weight):
    tab = jnp.transpose(weight.reshape(V, C, S), (0, 2, 1)).reshape(VS, C)
    out = _sc_gather(x.reshape(BV), tab)
    return out.reshape(B, V, 1, C)
